# Initial kernel scaffold; baseline (speedup 1.0000x reference)
import functools

import jax
import jax.numpy as jnp
from jax import lax
from jax.experimental import pallas as pl
from jax.experimental.pallas import tpu as pltpu
from jax.experimental.pallas import tpu_sc as plsc

VOCAB = 1000000
EMBED = 64
BATCH = 16384
NEG = 5

NC = 2
NS = 16
NW = NC * NS
BPW = BATCH // NW
CHUNK = 128
NCH = BPW // CHUNK
GROUPS = CHUNK // 16
NT = 1 + NEG
TROW = 17


def _sc_scores(center, context, neg_flat, w_center, w_context):
  mesh = plsc.VectorSubcoreMesh(core_axis_name="c", subcore_axis_name="s",
                                num_cores=NC, num_subcores=NS)

  @functools.partial(
      pl.kernel,
      out_type=jax.ShapeDtypeStruct((BATCH * NT,), jnp.float32),
      mesh=mesh,
      scratch_types=[
          pltpu.VMEM((CHUNK,), jnp.int32),
          pltpu.VMEM((CHUNK,), jnp.int32),
          pltpu.VMEM((CHUNK * NEG,), jnp.int32),
          pltpu.VMEM((CHUNK, EMBED), jnp.float32),
          pltpu.VMEM((CHUNK, EMBED), jnp.float32),
          pltpu.VMEM((CHUNK * NEG, EMBED), jnp.float32),
          pltpu.VMEM((NT * 16 * TROW,), jnp.float32),
          pltpu.VMEM((NT * CHUNK,), jnp.float32),
          pltpu.SemaphoreType.DMA,
      ],
  )
  def k(center_hbm, context_hbm, neg_hbm, wc_hbm, wx_hbm, out_hbm,
        cidx, xidx, nidx, crows, xrows, nrows, tscr, sbuf, sem):
    wid = lax.axis_index("s") * NC + lax.axis_index("c")
    lanes = lax.iota(jnp.int32, 16)

    def chunk_body(ch, carry):
      base = wid * BPW + ch * CHUNK
      pltpu.sync_copy(center_hbm.at[pl.ds(base, CHUNK)], cidx)
      pltpu.sync_copy(context_hbm.at[pl.ds(base, CHUNK)], xidx)
      pltpu.sync_copy(neg_hbm.at[pl.ds(base * NEG, CHUNK * NEG)], nidx)
      copies = [
          pltpu.async_copy(wc_hbm.at[cidx], crows, sem),
          pltpu.async_copy(wx_hbm.at[xidx], xrows, sem),
      ]
      for j in range(NEG):
        copies.append(pltpu.async_copy(
            wx_hbm.at[nidx.at[pl.ds(j * CHUNK, CHUNK)]],
            nrows.at[pl.ds(j * CHUNK, CHUNK)], sem))
      for cp in copies:
        cp.wait()

      def group_body(g, carry2):
        for e in range(16):
          b = g * 16 + e
          cvs = [crows[b, pl.ds(j * 16, 16)] for j in range(4)]
          xvs = [xrows[b, pl.ds(j * 16, 16)] for j in range(4)]
          p = cvs[0] * xvs[0]
          for j in range(1, 4):
            p = p + cvs[j] * xvs[j]
          tscr[pl.ds(0 * 16 * TROW + e * TROW, 16)] = p
          for t in range(NEG):
            nvs = [nrows[b * NEG + t, pl.ds(j * 16, 16)] for j in range(4)]
            q = cvs[0] * nvs[0]
            for j in range(1, 4):
              q = q + cvs[j] * nvs[j]
            tscr[pl.ds((t + 1) * 16 * TROW + e * TROW, 16)] = q
        for t in range(NT):
          s = plsc.load_gather(tscr, [lanes * TROW + t * 16 * TROW])
          for d in range(1, 16):
            s = s + plsc.load_gather(tscr, [lanes * TROW + (t * 16 * TROW + d)])
          if t > 0:
            s = -s
          sbuf[pl.ds(t * CHUNK + g * 16, 16)] = s
        return carry2

      lax.fori_loop(0, GROUPS, group_body, 0)
      pltpu.sync_copy(
          sbuf, out_hbm.at[pl.ds(wid * BPW * NT + ch * CHUNK * NT, CHUNK * NT)])
      return carry

    lax.fori_loop(0, NCH, chunk_body, 0)

  return k(center, context, neg_flat, w_center, w_context)


def _tc_loss(scores):
  x2 = scores.reshape(NT * BATCH // 128, 128)

  def body(x_ref, o_ref):
    x = x_ref[...]
    ls = jnp.minimum(x, 0.0) - jnp.log1p(jnp.exp(-jnp.abs(x)))
    o_ref[0, 0] = -jnp.sum(ls) / BATCH

  out = pl.pallas_call(
      body,
      out_shape=jax.ShapeDtypeStruct((1, 1), jnp.float32),
      out_specs=pl.BlockSpec(memory_space=pltpu.SMEM),
  )(x2)
  return out[0, 0]


@jax.jit
def kernel(center_words, context_words, negative_words, W_center, W_context):
  center = jnp.asarray(center_words, jnp.int32)
  context = jnp.asarray(context_words, jnp.int32)
  neg_flat = jnp.asarray(negative_words, jnp.int32).reshape(-1)
  scores = _sc_scores(center, context, neg_flat, W_center, W_context)
  return _tc_loss(scores)


# speedup vs baseline: 1.7106x; 1.7106x over previous
import functools

import jax
import jax.numpy as jnp
from jax import lax
from jax.experimental import pallas as pl
from jax.experimental.pallas import tpu as pltpu
from jax.experimental.pallas import tpu_sc as plsc

VOCAB = 1000000
EMBED = 64
BATCH = 16384
NEG = 5

NC = 2
NS = 16
NW = NC * NS
BPW = BATCH // NW
CHUNK = 128
NCH = BPW // CHUNK
GROUPS = CHUNK // 16
NT = 1 + NEG
TROW = 17


def _sc_scores(center, context, neg_flat, w_center, w_context):
  mesh = plsc.VectorSubcoreMesh(core_axis_name="c", subcore_axis_name="s",
                                num_cores=NC, num_subcores=NS)

  @functools.partial(
      pl.kernel,
      out_type=jax.ShapeDtypeStruct((BATCH * NT,), jnp.float32),
      mesh=mesh,
      compiler_params=pltpu.CompilerParams(needs_layout_passes=False,
                                           use_tc_tiling_on_sc=False),
      scratch_types=[
          pltpu.VMEM((CHUNK,), jnp.int32),
          pltpu.VMEM((CHUNK,), jnp.int32),
          pltpu.VMEM((CHUNK * NEG,), jnp.int32),
          pltpu.VMEM((CHUNK, EMBED), jnp.float32),
          pltpu.VMEM((CHUNK, EMBED), jnp.float32),
          pltpu.VMEM((CHUNK * NEG, EMBED), jnp.float32),
          pltpu.VMEM((NT * 16 * TROW,), jnp.float32),
          pltpu.VMEM((NT * CHUNK,), jnp.float32),
          pltpu.SemaphoreType.DMA,
      ],
  )
  def k(center_hbm, context_hbm, neg_hbm, wc_hbm, wx_hbm, out_hbm,
        cidx, xidx, nidx, crows, xrows, nrows, tscr, sbuf, sem):
    wid = lax.axis_index("s") * NC + lax.axis_index("c")
    lanes = lax.iota(jnp.int32, 16)

    def chunk_body(ch, carry):
      base = wid * BPW + ch * CHUNK
      pltpu.sync_copy(center_hbm.at[pl.ds(base, CHUNK)], cidx)
      pltpu.sync_copy(context_hbm.at[pl.ds(base, CHUNK)], xidx)
      pltpu.sync_copy(neg_hbm.at[pl.ds(base * NEG, CHUNK * NEG)], nidx)
      copies = [
          pltpu.async_copy(wc_hbm.at[cidx], crows, sem),
          pltpu.async_copy(wx_hbm.at[xidx], xrows, sem),
      ]
      for j in range(NEG):
        copies.append(pltpu.async_copy(
            wx_hbm.at[nidx.at[pl.ds(j * CHUNK, CHUNK)]],
            nrows.at[pl.ds(j * CHUNK, CHUNK)], sem))
      for cp in copies:
        cp.wait()

      def group_body(g, carry2):
        for e in range(16):
          b = g * 16 + e
          cvs = [crows[b, pl.ds(j * 16, 16)] for j in range(4)]
          xvs = [xrows[b, pl.ds(j * 16, 16)] for j in range(4)]
          p = cvs[0] * xvs[0]
          for j in range(1, 4):
            p = p + cvs[j] * xvs[j]
          tscr[pl.ds(0 * 16 * TROW + e * TROW, 16)] = p
          for t in range(NEG):
            nvs = [nrows[b * NEG + t, pl.ds(j * 16, 16)] for j in range(4)]
            q = cvs[0] * nvs[0]
            for j in range(1, 4):
              q = q + cvs[j] * nvs[j]
            tscr[pl.ds((t + 1) * 16 * TROW + e * TROW, 16)] = q
        for t in range(NT):
          s = plsc.load_gather(tscr, [lanes * TROW + t * 16 * TROW])
          for d in range(1, 16):
            s = s + plsc.load_gather(tscr, [lanes * TROW + (t * 16 * TROW + d)])
          if t > 0:
            s = -s
          sbuf[pl.ds(t * CHUNK + g * 16, 16)] = s
        return carry2

      lax.fori_loop(0, GROUPS, group_body, 0)
      pltpu.sync_copy(
          sbuf, out_hbm.at[pl.ds(wid * BPW * NT + ch * CHUNK * NT, CHUNK * NT)])
      return carry

    lax.fori_loop(0, NCH, chunk_body, 0)

  return k(center, context, neg_flat, w_center, w_context)


def _tc_loss(scores):
  x2 = scores.reshape(NT * BATCH // 128, 128)

  def body(x_ref, o_ref):
    x = x_ref[...]
    ls = jnp.minimum(x, 0.0) - jnp.log1p(jnp.exp(-jnp.abs(x)))
    o_ref[0, 0] = -jnp.sum(ls) / BATCH

  out = pl.pallas_call(
      body,
      out_shape=jax.ShapeDtypeStruct((1, 1), jnp.float32),
      out_specs=pl.BlockSpec(memory_space=pltpu.SMEM),
  )(x2)
  return out[0, 0]


@jax.jit
def kernel(center_words, context_words, negative_words, W_center, W_context):
  center = jnp.asarray(center_words, jnp.int32)
  context = jnp.asarray(context_words, jnp.int32)
  neg_flat = jnp.asarray(negative_words, jnp.int32).reshape(-1)
  scores = _sc_scores(center, context, neg_flat, W_center, W_context)
  return _tc_loss(scores)


# speedup vs baseline: 1.9404x; 1.1344x over previous
import functools

import jax
import jax.numpy as jnp
from jax import lax
from jax.experimental import pallas as pl
from jax.experimental.pallas import tpu as pltpu
from jax.experimental.pallas import tpu_sc as plsc

VOCAB = 1000000
EMBED = 64
BATCH = 16384
NEG = 5

NC = 2
NS = 16
NW = NC * NS
BPW = BATCH // NW
CHUNK = 128
NCH = BPW // CHUNK
GROUPS = CHUNK // 16
NT = 1 + NEG
TROW = 17
EPAD = 128
TBLK = 2048


def _tc_relayout(wt):
  grid = (VOCAB + TBLK - 1) // TBLK

  def body(x_ref, o_ref):
    o_ref[:, : EMBED] = x_ref[...].T

  return pl.pallas_call(
      body,
      grid=(grid,),
      in_specs=[pl.BlockSpec((EMBED, TBLK), lambda i: (0, i))],
      out_specs=pl.BlockSpec((TBLK, EPAD), lambda i: (i, 0)),
      out_shape=jax.ShapeDtypeStruct((VOCAB, EPAD), jnp.float32),
  )(wt)


def _sc_scores(center, context, neg_flat, w_center, w_context):
  mesh = plsc.VectorSubcoreMesh(core_axis_name="c", subcore_axis_name="s",
                                num_cores=NC, num_subcores=NS)

  @functools.partial(
      pl.kernel,
      out_type=jax.ShapeDtypeStruct((BATCH * NT,), jnp.float32),
      mesh=mesh,
      compiler_params=pltpu.CompilerParams(needs_layout_passes=False,
                                           use_tc_tiling_on_sc=True),
      scratch_types=[
          pltpu.VMEM((CHUNK,), jnp.int32),
          pltpu.VMEM((CHUNK,), jnp.int32),
          pltpu.VMEM((CHUNK * NEG,), jnp.int32),
          pltpu.VMEM((CHUNK, EPAD), jnp.float32),
          pltpu.VMEM((CHUNK, EPAD), jnp.float32),
          pltpu.VMEM((CHUNK * NEG, EPAD), jnp.float32),
          pltpu.VMEM((NT * 16 * TROW,), jnp.float32),
          pltpu.VMEM((NT * CHUNK,), jnp.float32),
          pltpu.SemaphoreType.DMA,
      ],
  )
  def k(center_hbm, context_hbm, neg_hbm, wc_hbm, wx_hbm, out_hbm,
        cidx, xidx, nidx, crows, xrows, nrows, tscr, sbuf, sem):
    wid = lax.axis_index("s") * NC + lax.axis_index("c")
    lanes = lax.iota(jnp.int32, 16)

    def chunk_body(ch, carry):
      base = wid * BPW + ch * CHUNK
      pltpu.sync_copy(center_hbm.at[pl.ds(base, CHUNK)], cidx)
      pltpu.sync_copy(context_hbm.at[pl.ds(base, CHUNK)], xidx)
      pltpu.sync_copy(neg_hbm.at[pl.ds(base * NEG, CHUNK * NEG)], nidx)
      copies = [
          pltpu.async_copy(wc_hbm.at[cidx], crows, sem),
          pltpu.async_copy(wx_hbm.at[xidx], xrows, sem),
      ]
      for j in range(NEG):
        copies.append(pltpu.async_copy(
            wx_hbm.at[nidx.at[pl.ds(j * CHUNK, CHUNK)]],
            nrows.at[pl.ds(j * CHUNK, CHUNK)], sem))
      for cp in copies:
        cp.wait()

      def group_body(g, carry2):
        for e in range(16):
          b = g * 16 + e
          cvs = [crows[b, pl.ds(j * 16, 16)] for j in range(4)]
          xvs = [xrows[b, pl.ds(j * 16, 16)] for j in range(4)]
          p = cvs[0] * xvs[0]
          for j in range(1, 4):
            p = p + cvs[j] * xvs[j]
          tscr[pl.ds(0 * 16 * TROW + e * TROW, 16)] = p
          for t in range(NEG):
            nvs = [nrows[b * NEG + t, pl.ds(j * 16, 16)] for j in range(4)]
            q = cvs[0] * nvs[0]
            for j in range(1, 4):
              q = q + cvs[j] * nvs[j]
            tscr[pl.ds((t + 1) * 16 * TROW + e * TROW, 16)] = q
        for t in range(NT):
          s = plsc.load_gather(tscr, [lanes * TROW + t * 16 * TROW])
          for d in range(1, 16):
            s = s + plsc.load_gather(tscr, [lanes * TROW + (t * 16 * TROW + d)])
          if t > 0:
            s = -s
          sbuf[pl.ds(t * CHUNK + g * 16, 16)] = s
        return carry2

      lax.fori_loop(0, GROUPS, group_body, 0)
      pltpu.sync_copy(
          sbuf, out_hbm.at[pl.ds(wid * BPW * NT + ch * CHUNK * NT, CHUNK * NT)])
      return carry

    lax.fori_loop(0, NCH, chunk_body, 0)

  return k(center, context, neg_flat, w_center, w_context)


def _tc_loss(scores):
  x2 = scores.reshape(NT * BATCH // 128, 128)

  def body(x_ref, o_ref):
    x = x_ref[...]
    ls = jnp.minimum(x, 0.0) - jnp.log1p(jnp.exp(-jnp.abs(x)))
    o_ref[0, 0] = -jnp.sum(ls) / BATCH

  out = pl.pallas_call(
      body,
      out_shape=jax.ShapeDtypeStruct((1, 1), jnp.float32),
      out_specs=pl.BlockSpec(memory_space=pltpu.SMEM),
  )(x2)
  return out[0, 0]


@jax.jit
def kernel(center_words, context_words, negative_words, W_center, W_context):
  center = jnp.asarray(center_words, jnp.int32)
  context = jnp.asarray(context_words, jnp.int32)
  neg_flat = jnp.asarray(negative_words, jnp.int32).reshape(-1)
  wc = _tc_relayout(W_center.T)
  wx = _tc_relayout(W_context.T)
  scores = _sc_scores(center, context, neg_flat, wc, wx)
  return _tc_loss(scores)


# speedup vs baseline: 2.0162x; 1.0390x over previous
import functools

import jax
import jax.numpy as jnp
from jax import lax
from jax.experimental import pallas as pl
from jax.experimental.pallas import tpu as pltpu
from jax.experimental.pallas import tpu_sc as plsc

VOCAB = 1000000
EMBED = 64
BATCH = 16384
NEG = 5

NC = 2
NS = 16
NW = NC * NS
BPW = BATCH // NW
CHUNK = 128
NCH = BPW // CHUNK
GROUPS = CHUNK // 16
NT = 1 + NEG
TROW = 17
EPAD = 128
TBLK = 2048


def _tc_relayout(wt):
  grid = (VOCAB + TBLK - 1) // TBLK
  tail = VOCAB - (grid - 1) * TBLK

  def body(x_ref, o_hbm, buf0, buf1, sem0, sem1):
    i = pl.program_id(0)

    def run(buf, sem):
      @pl.when(i >= 2)
      def _():
        pltpu.make_async_copy(
            buf, o_hbm.at[pl.ds((i - 2) * TBLK, TBLK)], sem).wait()

      buf[:, : EMBED] = x_ref[...].T

      @pl.when(i < grid - 1)
      def _():
        pltpu.make_async_copy(
            buf, o_hbm.at[pl.ds(i * TBLK, TBLK)], sem).start()

      @pl.when(i == grid - 1)
      def _():
        pltpu.make_async_copy(
            buf.at[pl.ds(0, tail)],
            o_hbm.at[pl.ds(i * TBLK, tail)], sem).start()

    @pl.when(i % 2 == 0)
    def _():
      run(buf0, sem0)

    @pl.when(i % 2 == 1)
    def _():
      run(buf1, sem1)

    @pl.when(i == grid - 1)
    def _():
      pltpu.make_async_copy(
          buf1, o_hbm.at[pl.ds((grid - 2) * TBLK, TBLK)], sem1).wait()
      pltpu.make_async_copy(
          buf0.at[pl.ds(0, tail)],
          o_hbm.at[pl.ds((grid - 1) * TBLK, tail)], sem0).wait()

  assert (grid - 1) % 2 == 0
  return pl.pallas_call(
      body,
      grid=(grid,),
      in_specs=[pl.BlockSpec((EMBED, TBLK), lambda i: (0, i))],
      out_specs=pl.BlockSpec(memory_space=pl.ANY),
      out_shape=jax.ShapeDtypeStruct((VOCAB, EPAD), jnp.float32),
      scratch_shapes=[
          pltpu.VMEM((TBLK, EPAD), jnp.float32),
          pltpu.VMEM((TBLK, EPAD), jnp.float32),
          pltpu.SemaphoreType.DMA,
          pltpu.SemaphoreType.DMA,
      ],
  )(wt)


def _sc_scores(center, context, neg_flat, w_center, w_context):
  mesh = plsc.VectorSubcoreMesh(core_axis_name="c", subcore_axis_name="s",
                                num_cores=NC, num_subcores=NS)

  @functools.partial(
      pl.kernel,
      out_type=jax.ShapeDtypeStruct((BATCH * NT,), jnp.float32),
      mesh=mesh,
      compiler_params=pltpu.CompilerParams(needs_layout_passes=False,
                                           use_tc_tiling_on_sc=True),
      scratch_types=[
          pltpu.VMEM((CHUNK,), jnp.int32),
          pltpu.VMEM((CHUNK,), jnp.int32),
          pltpu.VMEM((CHUNK * NEG,), jnp.int32),
          pltpu.VMEM((CHUNK, EPAD), jnp.float32),
          pltpu.VMEM((CHUNK, EPAD), jnp.float32),
          pltpu.VMEM((CHUNK * NEG, EPAD), jnp.float32),
          pltpu.VMEM((NT * 16 * TROW,), jnp.float32),
          pltpu.VMEM((NT * CHUNK,), jnp.float32),
          pltpu.SemaphoreType.DMA,
      ],
  )
  def k(center_hbm, context_hbm, neg_hbm, wc_hbm, wx_hbm, out_hbm,
        cidx, xidx, nidx, crows, xrows, nrows, tscr, sbuf, sem):
    wid = lax.axis_index("s") * NC + lax.axis_index("c")
    lanes = lax.iota(jnp.int32, 16)

    def chunk_body(ch, carry):
      base = wid * BPW + ch * CHUNK
      pltpu.sync_copy(center_hbm.at[pl.ds(base, CHUNK)], cidx)
      pltpu.sync_copy(context_hbm.at[pl.ds(base, CHUNK)], xidx)
      pltpu.sync_copy(neg_hbm.at[pl.ds(base * NEG, CHUNK * NEG)], nidx)
      copies = [
          pltpu.async_copy(wc_hbm.at[cidx], crows, sem),
          pltpu.async_copy(wx_hbm.at[xidx], xrows, sem),
      ]
      for j in range(NEG):
        copies.append(pltpu.async_copy(
            wx_hbm.at[nidx.at[pl.ds(j * CHUNK, CHUNK)]],
            nrows.at[pl.ds(j * CHUNK, CHUNK)], sem))
      for cp in copies:
        cp.wait()

      def group_body(g, carry2):
        for e in range(16):
          b = g * 16 + e
          cvs = [crows[b, pl.ds(j * 16, 16)] for j in range(4)]
          xvs = [xrows[b, pl.ds(j * 16, 16)] for j in range(4)]
          p = cvs[0] * xvs[0]
          for j in range(1, 4):
            p = p + cvs[j] * xvs[j]
          tscr[pl.ds(0 * 16 * TROW + e * TROW, 16)] = p
          for t in range(NEG):
            nvs = [nrows[b * NEG + t, pl.ds(j * 16, 16)] for j in range(4)]
            q = cvs[0] * nvs[0]
            for j in range(1, 4):
              q = q + cvs[j] * nvs[j]
            tscr[pl.ds((t + 1) * 16 * TROW + e * TROW, 16)] = q
        for t in range(NT):
          s = plsc.load_gather(tscr, [lanes * TROW + t * 16 * TROW])
          for d in range(1, 16):
            s = s + plsc.load_gather(tscr, [lanes * TROW + (t * 16 * TROW + d)])
          if t > 0:
            s = -s
          sbuf[pl.ds(t * CHUNK + g * 16, 16)] = s
        return carry2

      lax.fori_loop(0, GROUPS, group_body, 0)
      pltpu.sync_copy(
          sbuf, out_hbm.at[pl.ds(wid * BPW * NT + ch * CHUNK * NT, CHUNK * NT)])
      return carry

    lax.fori_loop(0, NCH, chunk_body, 0)

  return k(center, context, neg_flat, w_center, w_context)


def _tc_loss(scores):
  x2 = scores.reshape(NT * BATCH // 128, 128)

  def body(x_ref, o_ref):
    x = x_ref[...]
    ls = jnp.minimum(x, 0.0) - jnp.log1p(jnp.exp(-jnp.abs(x)))
    o_ref[0, 0] = -jnp.sum(ls) / BATCH

  out = pl.pallas_call(
      body,
      out_shape=jax.ShapeDtypeStruct((1, 1), jnp.float32),
      out_specs=pl.BlockSpec(memory_space=pltpu.SMEM),
  )(x2)
  return out[0, 0]


@jax.jit
def kernel(center_words, context_words, negative_words, W_center, W_context):
  center = jnp.asarray(center_words, jnp.int32)
  context = jnp.asarray(context_words, jnp.int32)
  neg_flat = jnp.asarray(negative_words, jnp.int32).reshape(-1)
  wc = _tc_relayout(W_center.T)
  wx = _tc_relayout(W_context.T)
  scores = _sc_scores(center, context, neg_flat, wc, wx)
  return _tc_loss(scores)
